# baseline (speedup 1.0000x reference)
import functools
import math

import jax
import jax.numpy as jnp
from jax import lax
from jax.experimental import pallas as pl
from jax.experimental.pallas import tpu as pltpu
from jax.experimental.pallas import tpu_sc as plsc

N = 10000
E = 320000
F = 128
C = 80
NC = 2
NS = 16
NW = NC * NS

_mesh = plsc.VectorSubcoreMesh(core_axis_name="c", subcore_axis_name="s")

EP = E // NW
NCH = EP // C

_SCRATCH_COMMON = [
    pltpu.VMEM((3, 2, C), jnp.int32),
    pltpu.VMEM((3, C), jnp.float32),
    pltpu.VMEM((C, F), jnp.float32),
    pltpu.VMEM((C, F), jnp.float32),
    pltpu.VMEM((C, F), jnp.float32),
    pltpu.VMEM((C, F), jnp.float32),
    pltpu.VMEM((2, C), jnp.int32),
    pltpu.VMEM_SHARED((N, F), jnp.float32),
    pltpu.SemaphoreType.DMA((3,)),
    pltpu.SemaphoreType.DMA,
    pltpu.SemaphoreType.DMA,
    pltpu.SemaphoreType.DMA,
    pltpu.SemaphoreType.DMA,
    pltpu.SemaphoreType.DMA,
]


@functools.partial(
    pl.kernel,
    out_type=jax.ShapeDtypeStruct((NC, N, F), jnp.float32),
    mesh=_mesh,
    scratch_types=_SCRATCH_COMMON,
)
def _spmm_sc(col_hbm, row_hbm, w_hbm, x_hbm, out_hbm,
             tbuf, wring, gbuf0, gbuf1, sbuf0, sbuf1, ridx, acc_sh,
             isem, gsem0, gsem1, ssem0, ssem1, osem):
    cid = lax.axis_index("c")
    sid = lax.axis_index("s")
    wid = sid * NC + cid
    ebase = wid * EP

    def _prefetch(j, s):
        sl = pl.ds(ebase + j * C, C)
        pltpu.async_copy(col_hbm.at[sl], tbuf.at[s, 0], isem.at[s])
        pltpu.async_copy(row_hbm.at[sl], tbuf.at[s, 1], isem.at[s])
        pltpu.async_copy(w_hbm.at[sl], wring.at[s], isem.at[s])

    def _wait_prefetch(j, s):
        sl = pl.ds(ebase + j * C, C)
        pltpu.make_async_copy(col_hbm.at[sl], tbuf.at[s, 0], isem.at[s]).wait()
        pltpu.make_async_copy(row_hbm.at[sl], tbuf.at[s, 1], isem.at[s]).wait()
        pltpu.make_async_copy(w_hbm.at[sl], wring.at[s], isem.at[s]).wait()

    def _issue_gather(s, gbuf, gsem):
        pltpu.async_copy(x_hbm.at[tbuf.at[s, 0]], gbuf, gsem)

    def _wait_gather(s, gbuf, gsem):
        pltpu.make_async_copy(x_hbm.at[tbuf.at[s, 0]], gbuf, gsem).wait()

    def _issue_scatter(sbuf, p, ssem):
        pltpu.async_copy(sbuf, acc_sh.at[ridx.at[p]], ssem, add=True)

    def _wait_scatter(sbuf, p, ssem):
        pltpu.make_async_copy(sbuf, acc_sh.at[ridx.at[p]], ssem).wait()

    _prefetch(0, 0)
    _prefetch(1, 1)
    _prefetch(2, 2)

    def _zrow(r, _):
        for k in range(F // 16):
            sbuf0[r, pl.ds(k * 16, 16)] = jnp.zeros((16,), jnp.float32)
        return 0
    lax.fori_loop(0, C, _zrow, 0)
    n_rchunk = N // C
    r_base = n_rchunk // NS
    r_extra = n_rchunk - r_base * NS
    r_count = r_base + jnp.where(sid < r_extra, 1, 0)

    def _zchunk(j, _):
        ch = sid + j * NS
        pltpu.async_copy(sbuf0, acc_sh.at[pl.ds(ch * C, C)], osem)
        return 0
    lax.fori_loop(0, r_count, _zchunk, 0)

    def _zdrain(j, _):
        pltpu.make_async_copy(sbuf0, acc_sh.at[pl.ds(sid * C, C)], osem).wait()
        return 0
    lax.fori_loop(0, r_count, _zdrain, 0)
    plsc.subcore_barrier()

    def _do_chunk(j, s, gbuf, sbuf, p, gsem, ssem, first=False, last=False):
        _wait_gather(s, gbuf, gsem)
        if not first:
            _wait_scatter(sbuf, p, ssem)

        def _grp(g, _):
            wvec = wring[s, pl.ds(g * 16, 16)]
            for l in range(16):
                ws = jnp.full((16,), wvec[l], jnp.float32)
                e = g * 16 + l
                for k in range(F // 16):
                    sl = pl.ds(k * 16, 16)
                    sbuf[e, sl] = gbuf[e, sl] * ws
            return 0
        lax.fori_loop(0, C // 16, _grp, 0, unroll=True)

        for g in range(C // 16):
            sl = pl.ds(g * 16, 16)
            ridx[p, sl] = tbuf[s, 1, sl]
        _issue_scatter(sbuf, p, ssem)
        if not last:
            s2 = jnp.where(s == 0, 2, s - 1)

            @pl.when(j + 2 < NCH)
            def _():
                _wait_prefetch(j + 2, s2)
                _issue_gather(s2, gbuf, gsem)

            @pl.when(j + 3 < NCH)
            def _():
                _prefetch(j + 3, s)

    _wait_prefetch(0, 0)
    _issue_gather(0, gbuf0, gsem0)
    _wait_prefetch(1, 1)
    _issue_gather(1, gbuf1, gsem1)

    _do_chunk(jnp.int32(0), jnp.int32(0), gbuf0, sbuf0, 0, gsem0, ssem0,
              first=True)
    _do_chunk(jnp.int32(1), jnp.int32(1), gbuf1, sbuf1, 1, gsem1, ssem1,
              first=True)

    def _pair(i, s):
        _do_chunk(2 * i, s, gbuf0, sbuf0, 0, gsem0, ssem0)
        s1 = jnp.where(s == 2, 0, s + 1)
        _do_chunk(2 * i + 1, s1, gbuf1, sbuf1, 1, gsem1, ssem1)
        return jnp.where(s1 == 2, 0, s1 + 1)
    lax.fori_loop(1, NCH // 2, _pair, jnp.int32(2))

    _do_chunk(jnp.int32(NCH - 1), jnp.int32((NCH - 1) % 3), gbuf0, sbuf0, 0,
              gsem0, ssem0, last=True)
    _wait_scatter(sbuf1, 1, ssem1)
    _wait_scatter(sbuf0, 0, ssem0)
    plsc.subcore_barrier()

    def _ochunk(j, _):
        ch = sid + j * NS
        pltpu.async_copy(acc_sh.at[pl.ds(ch * C, C)],
                         out_hbm.at[cid, pl.ds(ch * C, C)], osem)
        return 0
    lax.fori_loop(0, r_count, _ochunk, 0)

    def _odrain(j, _):
        pltpu.make_async_copy(acc_sh.at[pl.ds(sid * C, C)],
                              out_hbm.at[cid, pl.ds(sid * C, C)], osem).wait()
        return 0
    lax.fori_loop(0, r_count, _odrain, 0)


C2 = 40
NCH2 = EP // C2
_G2 = ((0, 0, 16), (16, 0, 16), (24, 8, 16))


@functools.partial(
    pl.kernel,
    out_type=jax.ShapeDtypeStruct((NC, N, F), jnp.float32),
    mesh=_mesh,
    scratch_types=[
        pltpu.VMEM((3, 2, C2), jnp.int32),
        pltpu.VMEM((3, C2), jnp.int32),
        pltpu.VMEM((3, C2), jnp.float32),
        pltpu.VMEM((C2, F), jnp.float32),
        pltpu.VMEM((C2, F), jnp.float32),
        pltpu.VMEM((C2, F), jnp.float32),
        pltpu.VMEM((C2, F), jnp.float32),
        pltpu.VMEM((C2, F), jnp.float32),
        pltpu.VMEM((C2, F), jnp.float32),
        pltpu.VMEM((2, C2), jnp.int32),
        pltpu.VMEM_SHARED((N, F), jnp.float32),
        pltpu.SemaphoreType.DMA((3,)),
        pltpu.SemaphoreType.DMA,
        pltpu.SemaphoreType.DMA,
        pltpu.SemaphoreType.DMA,
        pltpu.SemaphoreType.DMA,
        pltpu.SemaphoreType.DMA,
    ],
)
def _spmm_comb_sc(col_hbm, row_hbm, w_hbm, pf_hbm, out_hbm,
                  tbuf, oidx, wring, ga0, ga1, gb0, gb1, sbuf0, sbuf1, ridx,
                  acc_sh, isem, gsem0, gsem1, ssem0, ssem1, osem):
    cid = lax.axis_index("c")
    sid = lax.axis_index("s")
    wid = sid * NC + cid
    ebase = wid * EP

    def _prefetch(j, s):
        sl = pl.ds(ebase + j * C2, C2)
        pltpu.async_copy(col_hbm.at[sl], tbuf.at[s, 0], isem.at[s])
        pltpu.async_copy(row_hbm.at[sl], tbuf.at[s, 1], isem.at[s])
        pltpu.async_copy(w_hbm.at[sl], wring.at[s], isem.at[s])

    def _wait_prefetch(j, s):
        sl = pl.ds(ebase + j * C2, C2)
        pltpu.make_async_copy(col_hbm.at[sl], tbuf.at[s, 0], isem.at[s]).wait()
        pltpu.make_async_copy(row_hbm.at[sl], tbuf.at[s, 1], isem.at[s]).wait()
        pltpu.make_async_copy(w_hbm.at[sl], wring.at[s], isem.at[s]).wait()

    def _issue_gather(s, ga, gb, gsem):
        for off, _, n in _G2:
            sl = pl.ds(off, n)
            oidx[s, sl] = tbuf[s, 0, sl] + N
        pltpu.async_copy(pf_hbm.at[tbuf.at[s, 0]], ga, gsem)
        pltpu.async_copy(pf_hbm.at[oidx.at[s]], gb, gsem)

    def _wait_gather(s, ga, gb, gsem):
        pltpu.make_async_copy(pf_hbm.at[tbuf.at[s, 0]], ga, gsem).wait()
        pltpu.make_async_copy(pf_hbm.at[oidx.at[s]], gb, gsem).wait()

    def _issue_scatter(sbuf, p, ssem):
        pltpu.async_copy(sbuf, acc_sh.at[ridx.at[p]], ssem, add=True)

    def _wait_scatter(sbuf, p, ssem):
        pltpu.make_async_copy(sbuf, acc_sh.at[ridx.at[p]], ssem).wait()

    _prefetch(0, 0)
    _prefetch(1, 1)
    _prefetch(2, 2)

    def _zrow(r, _):
        for k in range(F // 16):
            sbuf0[r, pl.ds(k * 16, 16)] = jnp.zeros((16,), jnp.float32)
        return 0
    lax.fori_loop(0, C2, _zrow, 0)
    n_rchunk = N // C2
    r_base = n_rchunk // NS
    r_extra = n_rchunk - r_base * NS
    r_count = r_base + jnp.where(sid < r_extra, 1, 0)

    def _zchunk(j, _):
        ch = sid + j * NS
        pltpu.async_copy(sbuf0, acc_sh.at[pl.ds(ch * C2, C2)], osem)
        return 0
    lax.fori_loop(0, r_count, _zchunk, 0)

    def _zdrain(j, _):
        pltpu.make_async_copy(sbuf0, acc_sh.at[pl.ds(sid * C2, C2)],
                              osem).wait()
        return 0
    lax.fori_loop(0, r_count, _zdrain, 0)
    plsc.subcore_barrier()

    def _do_chunk(j, s, ga, gb, sbuf, p, gsem, ssem, first=False):
        _wait_gather(s, ga, gb, gsem)
        if not first:
            _wait_scatter(sbuf, p, ssem)

        for off, lo, n in _G2:
            wvec = wring[s, pl.ds(off, 16)]
            for l in range(lo, 16):
                ws = jnp.full((16,), wvec[l], jnp.float32)
                e = off + l
                for k in range(F // 16):
                    sl = pl.ds(k * 16, 16)
                    sbuf[e, sl] = (ga[e, sl] + gb[e, sl]) * ws

        for off, _, n in _G2:
            sl = pl.ds(off, n)
            ridx[p, sl] = tbuf[s, 1, sl]
        _issue_scatter(sbuf, p, ssem)
        s2 = jnp.where(s == 0, 2, s - 1)

        @pl.when(j + 2 < NCH2)
        def _():
            _wait_prefetch(j + 2, s2)
            _issue_gather(s2, ga, gb, gsem)

        @pl.when(j + 3 < NCH2)
        def _():
            _prefetch(j + 3, s)

    _wait_prefetch(0, 0)
    _issue_gather(0, ga0, gb0, gsem0)
    _wait_prefetch(1, 1)
    _issue_gather(1, ga1, gb1, gsem1)

    _do_chunk(jnp.int32(0), jnp.int32(0), ga0, gb0, sbuf0, 0, gsem0, ssem0,
              first=True)
    _do_chunk(jnp.int32(1), jnp.int32(1), ga1, gb1, sbuf1, 1, gsem1, ssem1,
              first=True)

    def _pair(i, s):
        _do_chunk(2 * i, s, ga0, gb0, sbuf0, 0, gsem0, ssem0)
        s1 = jnp.where(s == 2, 0, s + 1)
        _do_chunk(2 * i + 1, s1, ga1, gb1, sbuf1, 1, gsem1, ssem1)
        return jnp.where(s1 == 2, 0, s1 + 1)
    lax.fori_loop(1, NCH2 // 2, _pair, jnp.int32(2))

    _wait_scatter(sbuf0, 0, ssem0)
    _wait_scatter(sbuf1, 1, ssem1)
    plsc.subcore_barrier()

    def _ochunk(j, _):
        ch = sid + j * NS
        pltpu.async_copy(acc_sh.at[pl.ds(ch * C2, C2)],
                         out_hbm.at[cid, pl.ds(ch * C2, C2)], osem)
        return 0
    lax.fori_loop(0, r_count, _ochunk, 0)

    def _odrain(j, _):
        pltpu.make_async_copy(acc_sh.at[pl.ds(sid * C2, C2)],
                              out_hbm.at[cid, pl.ds(sid * C2, C2)],
                              osem).wait()
        return 0
    lax.fori_loop(0, r_count, _odrain, 0)


BM = 1000


def _tc_fin_body(x_ref, p1_ref, p2_ref, w0_ref, w1_ref, w2_ref, y_ref):
    inv_scale = 1.0 / math.sqrt(float(F))
    z1 = p1_ref[0] + p1_ref[1]
    z2 = p2_ref[0] + p2_ref[1]
    y_ref[...] = (
        jnp.dot(x_ref[...], w0_ref[...], preferred_element_type=jnp.float32)
        + jnp.dot(z1, w1_ref[...], preferred_element_type=jnp.float32)
        + jnp.dot(z2, w2_ref[...], preferred_element_type=jnp.float32)
    ) * inv_scale


def _tc_fin(x, p1, p2, W0, W1, W2):
    return pl.pallas_call(
        _tc_fin_body,
        grid=(N // BM,),
        in_specs=[
            pl.BlockSpec((BM, F), lambda i: (i, 0)),
            pl.BlockSpec((NC, BM, F), lambda i: (0, i, 0)),
            pl.BlockSpec((NC, BM, F), lambda i: (0, i, 0)),
            pl.BlockSpec((F, F), lambda i: (0, 0)),
            pl.BlockSpec((F, F), lambda i: (0, 0)),
            pl.BlockSpec((F, F), lambda i: (0, 0)),
        ],
        out_specs=pl.BlockSpec((BM, F), lambda i: (i, 0)),
        out_shape=jax.ShapeDtypeStruct((N, F), jnp.float32),
    )(x, p1, p2, W0, W1, W2)


def kernel(x, edge_index, edge_weight, W0, W1, W2):
    col = edge_index[1]
    row = edge_index[0]
    p1 = _spmm_sc(col, row, edge_weight, x)
    p2 = _spmm_comb_sc(col, row, edge_weight, p1.reshape(NC * N, F))
    return _tc_fin(x, p1, p2, W0, W1, W2)


# speedup vs baseline: 1.6613x; 1.6613x over previous
import functools
import math

import jax
import jax.numpy as jnp
from jax import lax
from jax.experimental import pallas as pl
from jax.experimental.pallas import tpu as pltpu
from jax.experimental.pallas import tpu_sc as plsc

N = 10000
E = 320000
F = 128
C = 80
NC = 2
NS = 16
NW = NC * NS
ROWS_PER_TILE = N // NS

_mesh = plsc.VectorSubcoreMesh(core_axis_name="c", subcore_axis_name="s")

EP = E // NW
NCH = EP // C


@functools.partial(
    pl.kernel,
    out_type=jax.ShapeDtypeStruct((NC, N, F), jnp.float32),
    mesh=_mesh,
    scratch_types=[
        pltpu.VMEM((3, 2, C), jnp.int32),
        pltpu.VMEM((3, C), jnp.float32),
        pltpu.VMEM((C, F), jnp.float32),
        pltpu.VMEM((C, F), jnp.float32),
        pltpu.VMEM((C, F), jnp.float32),
        pltpu.VMEM((C, F), jnp.float32),
        pltpu.VMEM((2, C), jnp.int32),
        pltpu.VMEM_SHARED((N, F), jnp.float32),
        pltpu.SemaphoreType.DMA((3,)),
        pltpu.SemaphoreType.DMA,
        pltpu.SemaphoreType.DMA,
        pltpu.SemaphoreType.DMA,
        pltpu.SemaphoreType.DMA,
        pltpu.SemaphoreType.DMA,
    ],
)
def _spmm_sc(col_hbm, row_hbm, w_hbm, x_hbm, out_hbm,
             tbuf, wring, gbuf0, gbuf1, sbuf0, sbuf1, ridx, acc_sh,
             isem, gsem0, gsem1, ssem0, ssem1, osem):
    cid = lax.axis_index("c")
    sid = lax.axis_index("s")
    wid = sid * NC + cid
    ebase = wid * EP

    def _prefetch(j, s):
        sl = pl.ds(ebase + j * C, C)
        pltpu.async_copy(col_hbm.at[sl], tbuf.at[s, 0], isem.at[s])
        pltpu.async_copy(row_hbm.at[sl], tbuf.at[s, 1], isem.at[s])
        pltpu.async_copy(w_hbm.at[sl], wring.at[s], isem.at[s])

    def _wait_prefetch(j, s):
        sl = pl.ds(ebase + j * C, C)
        pltpu.make_async_copy(col_hbm.at[sl], tbuf.at[s, 0], isem.at[s]).wait()
        pltpu.make_async_copy(row_hbm.at[sl], tbuf.at[s, 1], isem.at[s]).wait()
        pltpu.make_async_copy(w_hbm.at[sl], wring.at[s], isem.at[s]).wait()

    def _issue_gather(s, gbuf, gsem):
        pltpu.async_copy(x_hbm.at[tbuf.at[s, 0]], gbuf, gsem)

    def _wait_gather(s, gbuf, gsem):
        pltpu.make_async_copy(x_hbm.at[tbuf.at[s, 0]], gbuf, gsem).wait()

    def _issue_scatter(sbuf, p, ssem):
        pltpu.async_copy(sbuf, acc_sh.at[ridx.at[p]], ssem, add=True)

    def _wait_scatter(sbuf, p, ssem):
        pltpu.make_async_copy(sbuf, acc_sh.at[ridx.at[p]], ssem).wait()

    _prefetch(0, 0)
    _prefetch(1, 1)
    _prefetch(2, 2)

    def _zrow(r, _):
        for k in range(F // 16):
            sbuf0[r, pl.ds(k * 16, 16)] = jnp.zeros((16,), jnp.float32)
        return 0
    lax.fori_loop(0, C, _zrow, 0)
    n_rchunk = N // C
    r_base = n_rchunk // NS
    r_extra = n_rchunk - r_base * NS
    r_count = r_base + jnp.where(sid < r_extra, 1, 0)

    def _zchunk(j, _):
        ch = sid + j * NS
        pltpu.async_copy(sbuf0, acc_sh.at[pl.ds(ch * C, C)], osem)
        return 0
    lax.fori_loop(0, r_count, _zchunk, 0)

    def _zdrain(j, _):
        pltpu.make_async_copy(sbuf0, acc_sh.at[pl.ds(sid * C, C)], osem).wait()
        return 0
    lax.fori_loop(0, r_count, _zdrain, 0)
    plsc.subcore_barrier()

    def _do_chunk(j, s, gbuf, sbuf, p, gsem, ssem, first=False, last=False):
        _wait_gather(s, gbuf, gsem)
        if not first:
            _wait_scatter(sbuf, p, ssem)

        def _grp(g, _):
            wvec = wring[s, pl.ds(g * 16, 16)]
            for l in range(16):
                ws = jnp.full((16,), wvec[l], jnp.float32)
                e = g * 16 + l
                for k in range(F // 16):
                    sl = pl.ds(k * 16, 16)
                    sbuf[e, sl] = gbuf[e, sl] * ws
            return 0
        lax.fori_loop(0, C // 16, _grp, 0, unroll=True)

        for g in range(C // 16):
            sl = pl.ds(g * 16, 16)
            ridx[p, sl] = tbuf[s, 1, sl]
        _issue_scatter(sbuf, p, ssem)
        if not last:
            s2 = jnp.where(s == 0, 2, s - 1)

            @pl.when(j + 2 < NCH)
            def _():
                _wait_prefetch(j + 2, s2)
                _issue_gather(s2, gbuf, gsem)

            @pl.when(j + 3 < NCH)
            def _():
                _prefetch(j + 3, s)

    _wait_prefetch(0, 0)
    _issue_gather(0, gbuf0, gsem0)
    _wait_prefetch(1, 1)
    _issue_gather(1, gbuf1, gsem1)

    _do_chunk(jnp.int32(0), jnp.int32(0), gbuf0, sbuf0, 0, gsem0, ssem0,
              first=True)
    _do_chunk(jnp.int32(1), jnp.int32(1), gbuf1, sbuf1, 1, gsem1, ssem1,
              first=True)

    def _pair(i, s):
        _do_chunk(2 * i, s, gbuf0, sbuf0, 0, gsem0, ssem0)
        s1 = jnp.where(s == 2, 0, s + 1)
        _do_chunk(2 * i + 1, s1, gbuf1, sbuf1, 1, gsem1, ssem1)
        return jnp.where(s1 == 2, 0, s1 + 1)
    lax.fori_loop(1, NCH // 2, _pair, jnp.int32(2))

    _do_chunk(jnp.int32(NCH - 1), jnp.int32((NCH - 1) % 3), gbuf0, sbuf0, 0,
              gsem0, ssem0, last=True)
    _wait_scatter(sbuf1, 1, ssem1)
    _wait_scatter(sbuf0, 0, ssem0)
    plsc.subcore_barrier()

    def _ochunk(j, _):
        ch = sid + j * NS
        pltpu.async_copy(acc_sh.at[pl.ds(ch * C, C)],
                         out_hbm.at[cid, pl.ds(ch * C, C)], osem)
        return 0
    lax.fori_loop(0, r_count, _ochunk, 0)

    def _odrain(j, _):
        pltpu.make_async_copy(acc_sh.at[pl.ds(sid * C, C)],
                              out_hbm.at[cid, pl.ds(sid * C, C)], osem).wait()
        return 0
    lax.fori_loop(0, r_count, _odrain, 0)


BM = 1000


def _tc_add_body(p_ref, z_ref):
    z_ref[...] = p_ref[0] + p_ref[1]


def _tc_add(p):
    return pl.pallas_call(
        _tc_add_body,
        grid=(N // BM,),
        in_specs=[pl.BlockSpec((NC, BM, F), lambda i: (0, i, 0))],
        out_specs=pl.BlockSpec((BM, F), lambda i: (i, 0)),
        out_shape=jax.ShapeDtypeStruct((N, F), jnp.float32),
    )(p)


def _tc_fin_body(x_ref, z1_ref, p_ref, w0_ref, w1_ref, w2_ref, y_ref):
    inv_scale = 1.0 / math.sqrt(float(F))
    z2 = p_ref[0] + p_ref[1]
    y_ref[...] = (
        jnp.dot(x_ref[...], w0_ref[...], preferred_element_type=jnp.float32)
        + jnp.dot(z1_ref[...], w1_ref[...],
                  preferred_element_type=jnp.float32)
        + jnp.dot(z2, w2_ref[...], preferred_element_type=jnp.float32)
    ) * inv_scale


def _tc_fin(x, z1, p, W0, W1, W2):
    return pl.pallas_call(
        _tc_fin_body,
        grid=(N // BM,),
        in_specs=[
            pl.BlockSpec((BM, F), lambda i: (i, 0)),
            pl.BlockSpec((BM, F), lambda i: (i, 0)),
            pl.BlockSpec((NC, BM, F), lambda i: (0, i, 0)),
            pl.BlockSpec((F, F), lambda i: (0, 0)),
            pl.BlockSpec((F, F), lambda i: (0, 0)),
            pl.BlockSpec((F, F), lambda i: (0, 0)),
        ],
        out_specs=pl.BlockSpec((BM, F), lambda i: (i, 0)),
        out_shape=jax.ShapeDtypeStruct((N, F), jnp.float32),
    )(x, z1, p, W0, W1, W2)


def kernel(x, edge_index, edge_weight, W0, W1, W2):
    col = edge_index[1]
    row = edge_index[0]
    p1 = _spmm_sc(col, row, edge_weight, x)
    z1 = _tc_add(p1)
    p2 = _spmm_sc(col, row, edge_weight, z1)
    return _tc_fin(x, z1, p2, W0, W1, W2)
